# Initial kernel scaffold; baseline (speedup 1.0000x reference)
import jax
import jax.numpy as jnp
from jax.experimental import pallas as pl


def kernel(x, edge_index, W1, b1, W2, b2):
    raise NotImplementedError("write your pallas kernel here")


# speedup vs baseline: 37.0628x; 37.0628x over previous
import functools

import jax
import jax.numpy as jnp
from jax import lax
from jax.experimental import pallas as pl
from jax.experimental.pallas import tpu as pltpu
from jax.experimental.pallas import tpu_sc as plsc

N_NODES = 10000
N_EDGES = 320000
F_IN = 128
H = 16
C = 2

NP = 10240
EP_ROWS = 2560
EP = EP_ROWS * 128
NC = 2
NS = 16
RPT = EP_ROWS // (NC * NS)
NPT = NP // NS


def _sc_mesh():
    return plsc.VectorSubcoreMesh(core_axis_name="c", subcore_axis_name="s",
                                  num_cores=NC, num_subcores=NS)


_SC_PARAMS = pltpu.CompilerParams(use_tc_tiling_on_sc=False)


@functools.partial(
    pl.kernel,
    out_type=jax.ShapeDtypeStruct((NC, NP), jnp.float32),
    mesh=_sc_mesh(),
    compiler_params=_SC_PARAMS,
    scratch_types=[
        pltpu.VMEM((RPT, 128), jnp.int32),
        pltpu.VMEM((128,), jnp.float32),
        pltpu.VMEM((NPT,), jnp.float32),
        pltpu.VMEM_SHARED((NP,), jnp.float32),
    ],
)
def _deg_kernel(dst_hbm, out_hbm, idx_v, ones_v, buf_v, deg_sh):
    c = lax.axis_index("c")
    s = lax.axis_index("s")
    wid = s * NC + c

    def fill(i, _):
        buf_v[pl.ds(i * 16, 16)] = jnp.zeros((16,), jnp.float32)
        return 0
    lax.fori_loop(0, NPT // 16, fill, 0)
    for i in range(8):
        ones_v[pl.ds(i * 16, 16)] = jnp.ones((16,), jnp.float32)

    pltpu.sync_copy(buf_v, deg_sh.at[pl.ds(s * NPT, NPT)])
    pltpu.sync_copy(dst_hbm.at[pl.ds(wid * RPT, RPT)], idx_v)
    plsc.subcore_barrier()

    def body(j, _):
        pltpu.sync_copy(ones_v, deg_sh.at[idx_v.at[j]], add=True)
        return 0
    lax.fori_loop(0, RPT, body, 0)
    plsc.subcore_barrier()

    pltpu.sync_copy(deg_sh.at[pl.ds(s * NPT, NPT)], buf_v)
    pltpu.sync_copy(buf_v, out_hbm.at[c, pl.ds(s * NPT, NPT)])


def _make_agg_kernel(width):
    @functools.partial(
        pl.kernel,
        out_type=jax.ShapeDtypeStruct((NC, NP, width), jnp.float32),
        mesh=_sc_mesh(),
        compiler_params=_SC_PARAMS,
        scratch_types=[
            pltpu.VMEM((RPT, 128), jnp.int32),
            pltpu.VMEM((RPT, 128), jnp.int32),
            pltpu.VMEM((2, 128, width), jnp.float32),
            pltpu.VMEM((NPT, width), jnp.float32),
            pltpu.VMEM_SHARED((NP, width), jnp.float32),
            pltpu.SemaphoreType.DMA,
            pltpu.SemaphoreType.DMA,
        ],
    )
    def agg(src_hbm, dst_hbm, tab_hbm, zeros_hbm, out_hbm,
            src_v, dst_v, rows_v, buf_v, acc_sh, sem0, sem1):
        c = lax.axis_index("c")
        s = lax.axis_index("s")
        wid = s * NC + c

        pltpu.sync_copy(zeros_hbm.at[pl.ds(s * NPT, NPT)], buf_v)
        pltpu.sync_copy(buf_v, acc_sh.at[pl.ds(s * NPT, NPT)])
        pltpu.sync_copy(src_hbm.at[pl.ds(wid * RPT, RPT)], src_v)
        pltpu.sync_copy(dst_hbm.at[pl.ds(wid * RPT, RPT)], dst_v)
        plsc.subcore_barrier()

        def gather(j, slot, sem):
            return pltpu.async_copy(tab_hbm.at[src_v.at[j]], rows_v.at[slot],
                                    sem)

        def scat(j, slot):
            pltpu.sync_copy(rows_v.at[slot], acc_sh.at[dst_v.at[j]], add=True)

        gather(0, 0, sem0)

        def body(jo, _):
            j0 = 2 * jo
            gather(j0 + 1, 1, sem1)
            pltpu.make_async_copy(tab_hbm.at[src_v.at[j0]], rows_v.at[0],
                                  sem0).wait()
            scat(j0, 0)

            @pl.when(j0 + 2 < RPT)
            def _():
                gather(j0 + 2, 0, sem0)

            pltpu.make_async_copy(tab_hbm.at[src_v.at[j0 + 1]], rows_v.at[1],
                                  sem1).wait()
            scat(j0 + 1, 1)
            return 0
        lax.fori_loop(0, RPT // 2, body, 0)
        plsc.subcore_barrier()

        pltpu.sync_copy(acc_sh.at[pl.ds(s * NPT, NPT)], buf_v)
        pltpu.sync_copy(buf_v, out_hbm.at[c, pl.ds(s * NPT, NPT)])
    return agg


_agg16 = _make_agg_kernel(H)


_BLK = 512
_GRID = NP // _BLK


def _dinv_of(degp):
    deg = degp[0, :] + degp[1, :] + 1.0
    return lax.rsqrt(deg)


def _mm1_body(x_ref, w1_ref, degp_ref, out_ref):
    dinv = _dinv_of(degp_ref[...])
    h = jnp.dot(x_ref[...], w1_ref[...], preferred_element_type=jnp.float32)
    out_ref[...] = h * dinv[:, None]


def _mm1(x_p, W1, degp):
    return pl.pallas_call(
        _mm1_body,
        grid=(_GRID,),
        in_specs=[
            pl.BlockSpec((_BLK, F_IN), lambda i: (i, 0)),
            pl.BlockSpec((F_IN, H), lambda i: (0, 0)),
            pl.BlockSpec((NC, _BLK), lambda i: (0, i)),
        ],
        out_specs=pl.BlockSpec((_BLK, H), lambda i: (i, 0)),
        out_shape=jax.ShapeDtypeStruct((NP, H), jnp.float32),
    )(x_p, W1, degp)


def _mid_body(accA_ref, accB_ref, h1s_ref, degp_ref, b1_ref, out_ref):
    dinv = _dinv_of(degp_ref[...])
    agg = accA_ref[...] + accB_ref[...] + h1s_ref[...]
    out1 = agg * dinv[:, None] + b1_ref[...]
    t = jnp.maximum(out1, 0.0)
    out_ref[...] = t * dinv[:, None]


def _mid(accA, accB, h1s, degp, b1):
    return pl.pallas_call(
        _mid_body,
        grid=(_GRID,),
        in_specs=[
            pl.BlockSpec((_BLK, H), lambda i: (i, 0)),
            pl.BlockSpec((_BLK, H), lambda i: (i, 0)),
            pl.BlockSpec((_BLK, H), lambda i: (i, 0)),
            pl.BlockSpec((NC, _BLK), lambda i: (0, i)),
            pl.BlockSpec((1, H), lambda i: (0, 0)),
        ],
        out_specs=pl.BlockSpec((_BLK, H), lambda i: (i, 0)),
        out_shape=jax.ShapeDtypeStruct((NP, H), jnp.float32),
    )(accA, accB, h1s, degp, b1.reshape(1, H))


def _final_body(accA_ref, accB_ref, u_ref, degp_ref, w2_ref, b2_ref, out_ref):
    dinv = _dinv_of(degp_ref[...])
    agg = accA_ref[...] + accB_ref[...] + u_ref[...]
    h2 = jnp.dot(agg, w2_ref[...], preferred_element_type=jnp.float32)
    o = h2 * dinv[:, None] + b2_ref[...]
    m = jnp.max(o, axis=1, keepdims=True)
    z = o - m
    lse = jnp.log(jnp.exp(z[:, 0:1]) + jnp.exp(z[:, 1:2]))
    out_ref[...] = z - lse


def _final(accA, accB, u, degp, W2, b2):
    return pl.pallas_call(
        _final_body,
        grid=(_GRID,),
        in_specs=[
            pl.BlockSpec((_BLK, H), lambda i: (i, 0)),
            pl.BlockSpec((_BLK, H), lambda i: (i, 0)),
            pl.BlockSpec((_BLK, H), lambda i: (i, 0)),
            pl.BlockSpec((NC, _BLK), lambda i: (0, i)),
            pl.BlockSpec((H, C), lambda i: (0, 0)),
            pl.BlockSpec((1, C), lambda i: (0, 0)),
        ],
        out_specs=pl.BlockSpec((_BLK, C), lambda i: (i, 0)),
        out_shape=jax.ShapeDtypeStruct((NP, C), jnp.float32),
    )(accA, accB, u, degp, W2, b2.reshape(1, C))


def kernel(x, edge_index, W1, b1, W2, b2):
    ei = edge_index.astype(jnp.int32)
    pad = jnp.full((EP - N_EDGES,), N_NODES, jnp.int32)
    src_p = jnp.concatenate([ei[0], pad]).reshape(EP_ROWS, 128)
    dst_p = jnp.concatenate([ei[1], pad]).reshape(EP_ROWS, 128)
    x_p = jnp.pad(x, ((0, NP - N_NODES), (0, 0)))

    z16 = jnp.zeros((NP, H), jnp.float32)

    degp = _deg_kernel(dst_p)
    h1s = _mm1(x_p, W1, degp)
    acc = _agg16(src_p, dst_p, h1s, z16)
    u = _mid(acc[0], acc[1], h1s, degp, b1)
    acc2 = _agg16(src_p, dst_p, u, z16)
    out = _final(acc2[0], acc2[1], u, degp, W2, b2)
    return out[:N_NODES]


# speedup vs baseline: 38.6613x; 1.0431x over previous
import functools

import jax
import jax.numpy as jnp
from jax import lax
from jax.experimental import pallas as pl
from jax.experimental.pallas import tpu as pltpu
from jax.experimental.pallas import tpu_sc as plsc

N_NODES = 10000
N_EDGES = 320000
F_IN = 128
H = 16
C = 2

NP = 10240
EP = 327680
NC = 2
NS = 16
CH = 2048
NCH = EP // (NC * NS * CH)
NPT = NP // NS


def _sc_mesh():
    return plsc.VectorSubcoreMesh(core_axis_name="c", subcore_axis_name="s",
                                  num_cores=NC, num_subcores=NS)


_SC_PARAMS = pltpu.CompilerParams(use_tc_tiling_on_sc=False)


@functools.partial(
    pl.kernel,
    out_type=jax.ShapeDtypeStruct((NC, NP), jnp.float32),
    mesh=_sc_mesh(),
    compiler_params=_SC_PARAMS,
    scratch_types=[
        pltpu.VMEM((NCH, CH), jnp.int32),
        pltpu.VMEM((CH,), jnp.float32),
        pltpu.VMEM((NPT,), jnp.float32),
        pltpu.VMEM_SHARED((NP,), jnp.float32),
    ],
)
def _deg_kernel(dst_hbm, out_hbm, idx_v, ones_v, buf_v, deg_sh):
    c = lax.axis_index("c")
    s = lax.axis_index("s")
    wid = s * NC + c

    def fill(i, _):
        buf_v[pl.ds(i * 16, 16)] = jnp.zeros((16,), jnp.float32)
        return 0
    lax.fori_loop(0, NPT // 16, fill, 0)

    def fill1(i, _):
        ones_v[pl.ds(i * 16, 16)] = jnp.ones((16,), jnp.float32)
        return 0
    lax.fori_loop(0, CH // 16, fill1, 0)

    pltpu.sync_copy(buf_v, deg_sh.at[pl.ds(s * NPT, NPT)])
    pltpu.sync_copy(dst_hbm.at[pl.ds(wid * NCH, NCH)], idx_v)
    plsc.subcore_barrier()

    for j in range(NCH):
        pltpu.sync_copy(ones_v, deg_sh.at[idx_v.at[j]], add=True)
    plsc.subcore_barrier()

    pltpu.sync_copy(deg_sh.at[pl.ds(s * NPT, NPT)], buf_v)
    pltpu.sync_copy(buf_v, out_hbm.at[c, pl.ds(s * NPT, NPT)])


def _make_agg_kernel(width):
    @functools.partial(
        pl.kernel,
        out_type=jax.ShapeDtypeStruct((NC, NP, width), jnp.float32),
        mesh=_sc_mesh(),
        compiler_params=_SC_PARAMS,
        scratch_types=[
            pltpu.VMEM((NCH, CH), jnp.int32),
            pltpu.VMEM((NCH, CH), jnp.int32),
            pltpu.VMEM((2, CH, width), jnp.float32),
            pltpu.VMEM((NPT, width), jnp.float32),
            pltpu.VMEM_SHARED((NP, width), jnp.float32),
            pltpu.SemaphoreType.DMA,
            pltpu.SemaphoreType.DMA,
        ],
    )
    def agg(src_hbm, dst_hbm, tab_hbm, zeros_hbm, out_hbm,
            src_v, dst_v, rows_v, buf_v, acc_sh, sem0, sem1):
        c = lax.axis_index("c")
        s = lax.axis_index("s")
        wid = s * NC + c

        pltpu.sync_copy(zeros_hbm.at[pl.ds(s * NPT, NPT)], buf_v)
        pltpu.sync_copy(buf_v, acc_sh.at[pl.ds(s * NPT, NPT)])
        pltpu.sync_copy(src_hbm.at[pl.ds(wid * NCH, NCH)], src_v)
        pltpu.sync_copy(dst_hbm.at[pl.ds(wid * NCH, NCH)], dst_v)
        plsc.subcore_barrier()

        sems = [sem0, sem1]

        def gather(j):
            return pltpu.async_copy(tab_hbm.at[src_v.at[j]], rows_v.at[j % 2],
                                    sems[j % 2])

        desc = {0: gather(0)}
        for j in range(NCH):
            if j + 1 < NCH:
                desc[j + 1] = gather(j + 1)
            desc[j].wait()
            pltpu.sync_copy(rows_v.at[j % 2], acc_sh.at[dst_v.at[j]], add=True)
        plsc.subcore_barrier()

        pltpu.sync_copy(acc_sh.at[pl.ds(s * NPT, NPT)], buf_v)
        pltpu.sync_copy(buf_v, out_hbm.at[c, pl.ds(s * NPT, NPT)])
    return agg


_agg16 = _make_agg_kernel(H)


_BLK = 512
_GRID = NP // _BLK


def _dinv_of(degp):
    deg = degp[0, :] + degp[1, :] + 1.0
    return lax.rsqrt(deg)


def _mm1_body(x_ref, w1_ref, degp_ref, out_ref):
    dinv = _dinv_of(degp_ref[...])
    h = jnp.dot(x_ref[...], w1_ref[...], preferred_element_type=jnp.float32)
    out_ref[...] = h * dinv[:, None]


def _mm1(x_p, W1, degp):
    return pl.pallas_call(
        _mm1_body,
        grid=(_GRID,),
        in_specs=[
            pl.BlockSpec((_BLK, F_IN), lambda i: (i, 0)),
            pl.BlockSpec((F_IN, H), lambda i: (0, 0)),
            pl.BlockSpec((NC, _BLK), lambda i: (0, i)),
        ],
        out_specs=pl.BlockSpec((_BLK, H), lambda i: (i, 0)),
        out_shape=jax.ShapeDtypeStruct((NP, H), jnp.float32),
    )(x_p, W1, degp)


def _mid_body(accA_ref, accB_ref, h1s_ref, degp_ref, b1_ref, out_ref):
    dinv = _dinv_of(degp_ref[...])
    agg = accA_ref[...] + accB_ref[...] + h1s_ref[...]
    out1 = agg * dinv[:, None] + b1_ref[...]
    t = jnp.maximum(out1, 0.0)
    out_ref[...] = t * dinv[:, None]


def _mid(accA, accB, h1s, degp, b1):
    return pl.pallas_call(
        _mid_body,
        grid=(_GRID,),
        in_specs=[
            pl.BlockSpec((_BLK, H), lambda i: (i, 0)),
            pl.BlockSpec((_BLK, H), lambda i: (i, 0)),
            pl.BlockSpec((_BLK, H), lambda i: (i, 0)),
            pl.BlockSpec((NC, _BLK), lambda i: (0, i)),
            pl.BlockSpec((1, H), lambda i: (0, 0)),
        ],
        out_specs=pl.BlockSpec((_BLK, H), lambda i: (i, 0)),
        out_shape=jax.ShapeDtypeStruct((NP, H), jnp.float32),
    )(accA, accB, h1s, degp, b1.reshape(1, H))


def _final_body(accA_ref, accB_ref, u_ref, degp_ref, w2_ref, b2_ref, out_ref):
    dinv = _dinv_of(degp_ref[...])
    agg = accA_ref[...] + accB_ref[...] + u_ref[...]
    h2 = jnp.dot(agg, w2_ref[...], preferred_element_type=jnp.float32)
    o = h2 * dinv[:, None] + b2_ref[...]
    m = jnp.max(o, axis=1, keepdims=True)
    z = o - m
    lse = jnp.log(jnp.exp(z[:, 0:1]) + jnp.exp(z[:, 1:2]))
    out_ref[...] = z - lse


def _final(accA, accB, u, degp, W2, b2):
    return pl.pallas_call(
        _final_body,
        grid=(_GRID,),
        in_specs=[
            pl.BlockSpec((_BLK, H), lambda i: (i, 0)),
            pl.BlockSpec((_BLK, H), lambda i: (i, 0)),
            pl.BlockSpec((_BLK, H), lambda i: (i, 0)),
            pl.BlockSpec((NC, _BLK), lambda i: (0, i)),
            pl.BlockSpec((H, C), lambda i: (0, 0)),
            pl.BlockSpec((1, C), lambda i: (0, 0)),
        ],
        out_specs=pl.BlockSpec((_BLK, C), lambda i: (i, 0)),
        out_shape=jax.ShapeDtypeStruct((NP, C), jnp.float32),
    )(accA, accB, u, degp, W2, b2.reshape(1, C))


def kernel(x, edge_index, W1, b1, W2, b2):
    ei = edge_index.astype(jnp.int32)
    pad = jnp.full((EP - N_EDGES,), N_NODES, jnp.int32)
    src_p = jnp.concatenate([ei[0], pad]).reshape(NC * NS * NCH, CH)
    dst_p = jnp.concatenate([ei[1], pad]).reshape(NC * NS * NCH, CH)
    x_p = jnp.pad(x, ((0, NP - N_NODES), (0, 0)))

    z16 = jnp.zeros((NP, H), jnp.float32)

    degp = _deg_kernel(dst_p)
    h1s = _mm1(x_p, W1, degp)
    acc = _agg16(src_p, dst_p, h1s, z16)
    u = _mid(acc[0], acc[1], h1s, degp, b1)
    acc2 = _agg16(src_p, dst_p, u, z16)
    out = _final(acc2[0], acc2[1], u, degp, W2, b2)
    return out[:N_NODES]
